# Initial kernel scaffold; baseline (speedup 1.0000x reference)
import jax
import jax.numpy as jnp
from jax.experimental import pallas as pl


def kernel(x, edge_index, W1, b1, W2, b2, Wmu, bmu, Wls, bls):
    raise NotImplementedError("write your pallas kernel here")


# speedup vs baseline: 12.8065x; 12.8065x over previous
import functools

import jax
import jax.numpy as jnp
from jax import lax
from jax.experimental import pallas as pl
from jax.experimental.pallas import tpu as pltpu
from jax.experimental.pallas import tpu_sc as plsc

NN = 10000
EE = 320000
F = 128

NC = 2
NS = 16
NW = NC * NS
EPW = EE // NW
CHUNK = 80
NCHUNK = EPW // CHUNK
OUT_PAD = 10240
ROWS_PT = OUT_PAD // NS
RB = 128
DEG_PAD = 10240
DPT = DEG_PAD // NS


def _sc_deg_body(dst_hbm, out_hbm, ones_v, didx_v, zbuf_v, deg_sp, sem):
    c = lax.axis_index("c")
    s = lax.axis_index("s")
    w = c * NS + s
    for j in range(CHUNK // 16):
        ones_v[pl.ds(j * 16, 16)] = jnp.ones((16,), jnp.float32)
    for j in range(DPT // 16):
        zbuf_v[pl.ds(j * 16, 16)] = jnp.zeros((16,), jnp.float32)
    pltpu.sync_copy(zbuf_v, deg_sp.at[pl.ds(s * DPT, DPT)])
    plsc.subcore_barrier()

    def body(i, carry):
        base = w * EPW + i * CHUNK
        pltpu.sync_copy(dst_hbm.at[pl.ds(base, CHUNK)], didx_v)
        pltpu.sync_copy(ones_v, deg_sp.at[didx_v], add=True)
        return carry

    lax.fori_loop(0, NCHUNK, body, 0)
    plsc.subcore_barrier()
    pltpu.sync_copy(deg_sp.at[pl.ds(s * DPT, DPT)], zbuf_v)
    pltpu.sync_copy(zbuf_v, out_hbm.at[c, 0, pl.ds(s * DPT, DPT)])


def _sc_prop_body(g_hbm, src_hbm, dst_hbm, out_hbm,
                  sidx_v, didx_v, rows_v, zrow_v, agg_sp, sem):
    c = lax.axis_index("c")
    s = lax.axis_index("s")
    w = c * NS + s

    def zb(i, carry):
        for j in range(F // 16):
            zrow_v[i, pl.ds(j * 16, 16)] = jnp.zeros((16,), jnp.float32)
        return carry

    lax.fori_loop(0, RB, zb, 0)
    for r in range(ROWS_PT // RB):
        pltpu.sync_copy(zrow_v, agg_sp.at[pl.ds(s * ROWS_PT + r * RB, RB)])
    plsc.subcore_barrier()

    def body(i, carry):
        base = w * EPW + i * CHUNK
        pltpu.sync_copy(src_hbm.at[pl.ds(base, CHUNK)], sidx_v)
        pltpu.sync_copy(dst_hbm.at[pl.ds(base, CHUNK)], didx_v)
        pltpu.async_copy(g_hbm.at[sidx_v], rows_v, sem).wait()
        pltpu.sync_copy(rows_v, agg_sp.at[didx_v], add=True)
        return carry

    lax.fori_loop(0, NCHUNK, body, 0)
    plsc.subcore_barrier()
    for r in range(ROWS_PT // RB):
        pltpu.sync_copy(agg_sp.at[pl.ds(s * ROWS_PT + r * RB, RB)], zrow_v)
        pltpu.sync_copy(zrow_v, out_hbm.at[c, pl.ds(s * ROWS_PT + r * RB, RB)])


def _sc_mesh():
    return plsc.VectorSubcoreMesh(core_axis_name="c", subcore_axis_name="s")


_deg_call = pl.kernel(
    _sc_deg_body,
    out_type=jax.ShapeDtypeStruct((NC, 1, DEG_PAD), jnp.float32),
    mesh=_sc_mesh(),
    scratch_types=[
        pltpu.VMEM((CHUNK,), jnp.float32),
        pltpu.VMEM((CHUNK,), jnp.int32),
        pltpu.VMEM((DPT,), jnp.float32),
        pltpu.VMEM_SHARED((DEG_PAD,), jnp.float32),
        pltpu.SemaphoreType.DMA,
    ],
)

_prop_call = pl.kernel(
    _sc_prop_body,
    out_type=jax.ShapeDtypeStruct((NC, OUT_PAD, F), jnp.float32),
    mesh=_sc_mesh(),
    scratch_types=[
        pltpu.VMEM((CHUNK,), jnp.int32),
        pltpu.VMEM((CHUNK,), jnp.int32),
        pltpu.VMEM((CHUNK, F), jnp.float32),
        pltpu.VMEM((RB, F), jnp.float32),
        pltpu.VMEM_SHARED((OUT_PAD, F), jnp.float32),
        pltpu.SemaphoreType.DMA,
    ],
)


BR = 2000
GRID = NN // BR


def _mm1_body(x_ref, w_ref, d0_ref, d1_ref, g_ref, dis_ref):
    dis = lax.rsqrt(d0_ref[...] + d1_ref[...] + 1.0)
    u = jnp.dot(x_ref[...], w_ref[...], preferred_element_type=jnp.float32)
    dis_ref[...] = dis
    g_ref[...] = u * dis


def _comb_mm_body(p_ref, g_ref, dis_ref, b_ref, w_ref, out_ref):
    dis = dis_ref[...]
    a = (p_ref[0] + p_ref[1] + g_ref[...]) * dis + b_ref[...]
    h = jnp.maximum(a, 0.0)
    out_ref[...] = jnp.dot(h, w_ref[...], preferred_element_type=jnp.float32) * dis


def _comb_body(p_ref, g_ref, dis_ref, b_ref, out_ref):
    dis = dis_ref[...]
    a = (p_ref[0] + p_ref[1] + g_ref[...]) * dis + b_ref[...]
    out_ref[...] = jnp.maximum(a, 0.0) * dis


def _final_body(p_ref, g_ref, dis_ref, w_ref, b_ref, out_ref):
    p = (p_ref[0] + p_ref[1] + g_ref[...]) * dis_ref[...]
    out_ref[...] = jnp.dot(p, w_ref[...], preferred_element_type=jnp.float32) + b_ref[...]


_row_spec = pl.BlockSpec((BR, F), lambda i: (i, 0))
_p_spec = pl.BlockSpec((NC, BR, F), lambda i: (0, i, 0))
_dis_spec = pl.BlockSpec((BR, 1), lambda i: (i, 0))
_w_spec = pl.BlockSpec((F, F), lambda i: (0, 0))
_b_spec = pl.BlockSpec((1, F), lambda i: (0, 0))

_mm1 = pl.pallas_call(
    _mm1_body,
    grid=(GRID,),
    in_specs=[_row_spec, _w_spec, _dis_spec, _dis_spec],
    out_specs=[_row_spec, _dis_spec],
    out_shape=[jax.ShapeDtypeStruct((NN, F), jnp.float32),
               jax.ShapeDtypeStruct((NN, 1), jnp.float32)],
)

_comb_mm = pl.pallas_call(
    _comb_mm_body,
    grid=(GRID,),
    in_specs=[_p_spec, _row_spec, _dis_spec, _b_spec, _w_spec],
    out_specs=_row_spec,
    out_shape=jax.ShapeDtypeStruct((NN, F), jnp.float32),
)

_comb = pl.pallas_call(
    _comb_body,
    grid=(GRID,),
    in_specs=[_p_spec, _row_spec, _dis_spec, _b_spec],
    out_specs=_row_spec,
    out_shape=jax.ShapeDtypeStruct((NN, F), jnp.float32),
)

_final = pl.pallas_call(
    _final_body,
    grid=(GRID,),
    in_specs=[_p_spec, _row_spec, _dis_spec, _w_spec, _b_spec],
    out_specs=_row_spec,
    out_shape=jax.ShapeDtypeStruct((NN, F), jnp.float32),
)


def kernel(x, edge_index, W1, b1, W2, b2, Wmu, bmu, Wls, bls):
    src = edge_index[0]
    dst = edge_index[1]

    degp = _deg_call(dst)
    d0 = degp[0, 0, :NN, None]
    d1 = degp[1, 0, :NN, None]

    g1, dis = _mm1(x, W1, d0, d1)
    P1 = _prop_call(g1, src, dst)
    g2 = _comb_mm(P1, g1, dis, b1[None], W2)
    P2 = _prop_call(g2, src, dst)
    g3 = _comb(P2, g2, dis, b2[None])
    P3 = _prop_call(g3, src, dst)

    Wc = jnp.concatenate([Wmu, Wls], axis=1)
    bc = jnp.concatenate([bmu, bls])[None]
    out = _final(P3, g3, dis, Wc, bc)
    return (out[:, :64], out[:, 64:])


# speedup vs baseline: 19.0110x; 1.4845x over previous
import functools

import jax
import jax.numpy as jnp
from jax import lax
from jax.experimental import pallas as pl
from jax.experimental.pallas import tpu as pltpu
from jax.experimental.pallas import tpu_sc as plsc

NN = 10000
EE = 320000
F = 128

NC = 2
NS = 16
NW = NC * NS
EPW = EE // NW
CHUNK = 80
NCHP = EPW // CHUNK
OUT_PAD = 10240
ROWS_PT = OUT_PAD // NS
RB = 128
NCHD = EPW // CHUNK
DEG_PAD = 10240
DPT = DEG_PAD // NS


def _sc_deg_body(dst_hbm, out_hbm, ones_v, didx_v, zbuf_v, deg_sp, sem):
    c = lax.axis_index("c")
    s = lax.axis_index("s")
    w = c * NS + s
    for j in range(CHUNK // 16):
        ones_v[pl.ds(j * 16, 16)] = jnp.ones((16,), jnp.float32)
    for j in range(DPT // 16):
        zbuf_v[pl.ds(j * 16, 16)] = jnp.zeros((16,), jnp.float32)
    pltpu.sync_copy(zbuf_v, deg_sp.at[pl.ds(s * DPT, DPT)])
    plsc.subcore_barrier()

    def body(i, carry):
        base = w * EPW + i * CHUNK
        pltpu.sync_copy(dst_hbm.at[pl.ds(base, CHUNK)], didx_v)
        pltpu.sync_copy(ones_v, deg_sp.at[didx_v], add=True)
        return carry

    lax.fori_loop(0, NCHD, body, 0)
    plsc.subcore_barrier()
    pltpu.sync_copy(deg_sp.at[pl.ds(s * DPT, DPT)], zbuf_v)
    pltpu.sync_copy(zbuf_v, out_hbm.at[c, 0, pl.ds(s * DPT, DPT)])


def _sc_prop_body(g_hbm, src4_hbm, dst4_hbm, out_hbm,
                  sa_v, da_v, sb_v, db_v, rowsa_v, rowsb_v, zrow_v, agg_sp,
                  gsem):
    c = lax.axis_index("c")
    s = lax.axis_index("s")
    w = c * NS + s

    def zb(i, carry):
        for j in range(F // 16):
            zrow_v[i, pl.ds(j * 16, 16)] = jnp.zeros((16,), jnp.float32)
        return carry

    lax.fori_loop(0, RB, zb, 0)
    for r in range(ROWS_PT // RB):
        pltpu.sync_copy(zrow_v, agg_sp.at[pl.ds(s * ROWS_PT + r * RB, RB)])
    plsc.subcore_barrier()

    def gather(si, rows):
        pltpu.async_copy(g_hbm.at[si], rows, gsem)

    def gather_wait(si, rows):
        pltpu.make_async_copy(g_hbm.at[si], rows, gsem).wait()

    pltpu.sync_copy(src4_hbm.at[w, 0], sa_v)
    pltpu.sync_copy(dst4_hbm.at[w, 0], da_v)
    gather(sa_v, rowsa_v)

    def body(k, carry):
        jb = 2 * k + 1
        pltpu.sync_copy(src4_hbm.at[w, jb], sb_v)
        pltpu.sync_copy(dst4_hbm.at[w, jb], db_v)
        gather(sb_v, rowsb_v)
        gather_wait(sa_v, rowsa_v)
        pltpu.sync_copy(rowsa_v, agg_sp.at[da_v], add=True)
        pltpu.sync_copy(src4_hbm.at[w, jb + 1], sa_v)
        pltpu.sync_copy(dst4_hbm.at[w, jb + 1], da_v)
        gather(sa_v, rowsa_v)
        gather_wait(sb_v, rowsb_v)
        pltpu.sync_copy(rowsb_v, agg_sp.at[db_v], add=True)
        return carry

    lax.fori_loop(0, (NCHP - 1) // 2, body, 0)
    gather_wait(sa_v, rowsa_v)
    pltpu.sync_copy(rowsa_v, agg_sp.at[da_v], add=True)
    plsc.subcore_barrier()
    for r in range(ROWS_PT // RB):
        pltpu.sync_copy(agg_sp.at[pl.ds(s * ROWS_PT + r * RB, RB)], zrow_v)
        pltpu.sync_copy(zrow_v, out_hbm.at[c, pl.ds(s * ROWS_PT + r * RB, RB)])


def _sc_mesh():
    return plsc.VectorSubcoreMesh(core_axis_name="c", subcore_axis_name="s")


_deg_call = pl.kernel(
    _sc_deg_body,
    out_type=jax.ShapeDtypeStruct((NC, 1, DEG_PAD), jnp.float32),
    mesh=_sc_mesh(),
    scratch_types=[
        pltpu.VMEM((CHUNK,), jnp.float32),
        pltpu.VMEM((CHUNK,), jnp.int32),
        pltpu.VMEM((DPT,), jnp.float32),
        pltpu.VMEM_SHARED((DEG_PAD,), jnp.float32),
        pltpu.SemaphoreType.DMA,
    ],
)

_prop_call = pl.kernel(
    _sc_prop_body,
    out_type=jax.ShapeDtypeStruct((NC, OUT_PAD, F), jnp.float32),
    mesh=_sc_mesh(),
    scratch_types=[
        pltpu.VMEM((CHUNK,), jnp.int32),
        pltpu.VMEM((CHUNK,), jnp.int32),
        pltpu.VMEM((CHUNK,), jnp.int32),
        pltpu.VMEM((CHUNK,), jnp.int32),
        pltpu.VMEM((CHUNK, F), jnp.float32),
        pltpu.VMEM((CHUNK, F), jnp.float32),
        pltpu.VMEM((RB, F), jnp.float32),
        pltpu.VMEM_SHARED((OUT_PAD, F), jnp.float32),
        pltpu.SemaphoreType.DMA,
    ],
)


BR = 2000
GRID = NN // BR


def _mm1_body(x_ref, w_ref, d0_ref, d1_ref, g_ref, dis_ref):
    dis = lax.rsqrt(d0_ref[...] + d1_ref[...] + 1.0)
    u = jnp.dot(x_ref[...], w_ref[...], preferred_element_type=jnp.float32)
    dis_ref[...] = dis
    g_ref[...] = u * dis


def _comb_mm_body(p_ref, g_ref, dis_ref, b_ref, w_ref, out_ref):
    dis = dis_ref[...]
    a = (p_ref[0] + p_ref[1] + g_ref[...]) * dis + b_ref[...]
    h = jnp.maximum(a, 0.0)
    out_ref[...] = jnp.dot(h, w_ref[...], preferred_element_type=jnp.float32) * dis


def _comb_body(p_ref, g_ref, dis_ref, b_ref, out_ref):
    dis = dis_ref[...]
    a = (p_ref[0] + p_ref[1] + g_ref[...]) * dis + b_ref[...]
    out_ref[...] = jnp.maximum(a, 0.0) * dis


def _final_body(p_ref, g_ref, dis_ref, w_ref, b_ref, out_ref):
    p = (p_ref[0] + p_ref[1] + g_ref[...]) * dis_ref[...]
    out_ref[...] = jnp.dot(p, w_ref[...], preferred_element_type=jnp.float32) + b_ref[...]


_row_spec = pl.BlockSpec((BR, F), lambda i: (i, 0))
_p_spec = pl.BlockSpec((NC, BR, F), lambda i: (0, i, 0))
_dis_spec = pl.BlockSpec((BR, 1), lambda i: (i, 0))
_w_spec = pl.BlockSpec((F, F), lambda i: (0, 0))
_b_spec = pl.BlockSpec((1, F), lambda i: (0, 0))

_mm1 = pl.pallas_call(
    _mm1_body,
    grid=(GRID,),
    in_specs=[_row_spec, _w_spec, _dis_spec, _dis_spec],
    out_specs=[_row_spec, _dis_spec],
    out_shape=[jax.ShapeDtypeStruct((NN, F), jnp.float32),
               jax.ShapeDtypeStruct((NN, 1), jnp.float32)],
)

_comb_mm = pl.pallas_call(
    _comb_mm_body,
    grid=(GRID,),
    in_specs=[_p_spec, _row_spec, _dis_spec, _b_spec, _w_spec],
    out_specs=_row_spec,
    out_shape=jax.ShapeDtypeStruct((NN, F), jnp.float32),
)

_comb = pl.pallas_call(
    _comb_body,
    grid=(GRID,),
    in_specs=[_p_spec, _row_spec, _dis_spec, _b_spec],
    out_specs=_row_spec,
    out_shape=jax.ShapeDtypeStruct((NN, F), jnp.float32),
)

_final = pl.pallas_call(
    _final_body,
    grid=(GRID,),
    in_specs=[_p_spec, _row_spec, _dis_spec, _w_spec, _b_spec],
    out_specs=_row_spec,
    out_shape=jax.ShapeDtypeStruct((NN, F), jnp.float32),
)


def kernel(x, edge_index, W1, b1, W2, b2, Wmu, bmu, Wls, bls):
    src = edge_index[0]
    dst = edge_index[1]
    srcr = src.reshape(NW, NCHP, CHUNK)
    dstr = dst.reshape(NW, NCHP, CHUNK)

    degp = _deg_call(dst)
    d0 = degp[0, 0, :NN, None]
    d1 = degp[1, 0, :NN, None]

    g1, dis = _mm1(x, W1, d0, d1)
    P1 = _prop_call(g1, srcr, dstr)
    g2 = _comb_mm(P1, g1, dis, b1[None], W2)
    P2 = _prop_call(g2, srcr, dstr)
    g3 = _comb(P2, g2, dis, b2[None])
    P3 = _prop_call(g3, srcr, dstr)

    Wc = jnp.concatenate([Wmu, Wls], axis=1)
    bc = jnp.concatenate([bmu, bls])[None]
    out = _final(P3, g3, dis, Wc, bc)
    return (out[:, :64], out[:, 64:])


# speedup vs baseline: 23.1402x; 1.2172x over previous
import functools

import jax
import jax.numpy as jnp
from jax import lax
from jax.experimental import pallas as pl
from jax.experimental.pallas import tpu as pltpu
from jax.experimental.pallas import tpu_sc as plsc

NN = 10000
EE = 320000
F = 128

NC = 2
NS = 16
NW = NC * NS
EPW = EE // NW
CHUNK = 80
NCHP = EPW // CHUNK
OUT_PAD = 10240
ROWS_PT = OUT_PAD // NS
RB = 128
NCHD = EPW // CHUNK
DEG_PAD = 10240
DPT = DEG_PAD // NS


def _sc_deg_body(dst_hbm, out_hbm, ones_v, didx_v, zbuf_v, deg_sp, sem):
    c = lax.axis_index("c")
    s = lax.axis_index("s")
    w = c * NS + s
    for j in range(CHUNK // 16):
        ones_v[pl.ds(j * 16, 16)] = jnp.ones((16,), jnp.float32)
    for j in range(DPT // 16):
        zbuf_v[pl.ds(j * 16, 16)] = jnp.zeros((16,), jnp.float32)
    pltpu.sync_copy(zbuf_v, deg_sp.at[pl.ds(s * DPT, DPT)])
    plsc.subcore_barrier()

    def body(i, carry):
        base = w * EPW + i * CHUNK
        pltpu.sync_copy(dst_hbm.at[pl.ds(base, CHUNK)], didx_v)
        pltpu.sync_copy(ones_v, deg_sp.at[didx_v], add=True)
        return carry

    lax.fori_loop(0, NCHD, body, 0)
    plsc.subcore_barrier()
    pltpu.sync_copy(deg_sp.at[pl.ds(s * DPT, DPT)], zbuf_v)
    pltpu.sync_copy(zbuf_v, out_hbm.at[c, 0, pl.ds(s * DPT, DPT)])


def _sc_prop_body(g_hbm, e4_hbm, out_hbm,
                  ea_v, eb_v, rowsa_v, rowsb_v, zrow_v, agg_sp, gsem):
    c = lax.axis_index("c")
    s = lax.axis_index("s")
    w = c * NS + s

    def zb(i, carry):
        for j in range(F // 16):
            zrow_v[i, pl.ds(j * 16, 16)] = jnp.zeros((16,), jnp.float32)
        return carry

    lax.fori_loop(0, RB, zb, 0)
    for r in range(ROWS_PT // RB):
        pltpu.sync_copy(zrow_v, agg_sp.at[pl.ds(s * ROWS_PT + r * RB, RB)])
    plsc.subcore_barrier()

    def gather(si, rows):
        pltpu.async_copy(g_hbm.at[si], rows, gsem)

    def gather_wait(si, rows):
        pltpu.make_async_copy(g_hbm.at[si], rows, gsem).wait()

    pltpu.sync_copy(e4_hbm.at[w, 0], ea_v)
    gather(ea_v.at[0], rowsa_v)

    def body(k, carry):
        jb = 2 * k + 1
        pltpu.sync_copy(e4_hbm.at[w, jb], eb_v)
        gather(eb_v.at[0], rowsb_v)
        gather_wait(ea_v.at[0], rowsa_v)
        pltpu.sync_copy(rowsa_v, agg_sp.at[ea_v.at[1]], add=True)
        pltpu.sync_copy(e4_hbm.at[w, jb + 1], ea_v)
        gather(ea_v.at[0], rowsa_v)
        gather_wait(eb_v.at[0], rowsb_v)
        pltpu.sync_copy(rowsb_v, agg_sp.at[eb_v.at[1]], add=True)
        return carry

    lax.fori_loop(0, (NCHP - 1) // 2, body, 0)
    gather_wait(ea_v.at[0], rowsa_v)
    pltpu.sync_copy(rowsa_v, agg_sp.at[ea_v.at[1]], add=True)
    plsc.subcore_barrier()
    for r in range(ROWS_PT // RB):
        pltpu.sync_copy(agg_sp.at[pl.ds(s * ROWS_PT + r * RB, RB)], zrow_v)
        pltpu.sync_copy(zrow_v, out_hbm.at[c, pl.ds(s * ROWS_PT + r * RB, RB)])


def _sc_mesh():
    return plsc.VectorSubcoreMesh(core_axis_name="c", subcore_axis_name="s")


_deg_call = pl.kernel(
    _sc_deg_body,
    out_type=jax.ShapeDtypeStruct((NC, 1, DEG_PAD), jnp.float32),
    mesh=_sc_mesh(),
    scratch_types=[
        pltpu.VMEM((CHUNK,), jnp.float32),
        pltpu.VMEM((CHUNK,), jnp.int32),
        pltpu.VMEM((DPT,), jnp.float32),
        pltpu.VMEM_SHARED((DEG_PAD,), jnp.float32),
        pltpu.SemaphoreType.DMA,
    ],
)

_prop_call = pl.kernel(
    _sc_prop_body,
    out_type=jax.ShapeDtypeStruct((NC, OUT_PAD, F), jnp.float32),
    mesh=_sc_mesh(),
    scratch_types=[
        pltpu.VMEM((2, CHUNK), jnp.int32),
        pltpu.VMEM((2, CHUNK), jnp.int32),
        pltpu.VMEM((CHUNK, F), jnp.float32),
        pltpu.VMEM((CHUNK, F), jnp.float32),
        pltpu.VMEM((RB, F), jnp.float32),
        pltpu.VMEM_SHARED((OUT_PAD, F), jnp.float32),
        pltpu.SemaphoreType.DMA,
    ],
)


BR = 2000
GRID = NN // BR


def _mm1_body(x_ref, w_ref, d0_ref, d1_ref, g_ref, dis_ref):
    dis = lax.rsqrt(d0_ref[...] + d1_ref[...] + 1.0)
    u = jnp.dot(x_ref[...], w_ref[...], preferred_element_type=jnp.float32)
    dis_ref[...] = dis
    g_ref[...] = u * dis


def _comb_mm_body(p_ref, g_ref, dis_ref, b_ref, w_ref, out_ref):
    dis = dis_ref[...]
    a = (p_ref[0] + p_ref[1] + g_ref[...]) * dis + b_ref[...]
    h = jnp.maximum(a, 0.0)
    out_ref[...] = jnp.dot(h, w_ref[...], preferred_element_type=jnp.float32) * dis


def _comb_body(p_ref, g_ref, dis_ref, b_ref, out_ref):
    dis = dis_ref[...]
    a = (p_ref[0] + p_ref[1] + g_ref[...]) * dis + b_ref[...]
    out_ref[...] = jnp.maximum(a, 0.0) * dis


def _final_body(p_ref, g_ref, dis_ref, w_ref, b_ref, out_ref):
    p = (p_ref[0] + p_ref[1] + g_ref[...]) * dis_ref[...]
    out_ref[...] = jnp.dot(p, w_ref[...], preferred_element_type=jnp.float32) + b_ref[...]


_row_spec = pl.BlockSpec((BR, F), lambda i: (i, 0))
_p_spec = pl.BlockSpec((NC, BR, F), lambda i: (0, i, 0))
_dis_spec = pl.BlockSpec((BR, 1), lambda i: (i, 0))
_w_spec = pl.BlockSpec((F, F), lambda i: (0, 0))
_b_spec = pl.BlockSpec((1, F), lambda i: (0, 0))

_mm1 = pl.pallas_call(
    _mm1_body,
    grid=(GRID,),
    in_specs=[_row_spec, _w_spec, _dis_spec, _dis_spec],
    out_specs=[_row_spec, _dis_spec],
    out_shape=[jax.ShapeDtypeStruct((NN, F), jnp.float32),
               jax.ShapeDtypeStruct((NN, 1), jnp.float32)],
)

_comb_mm = pl.pallas_call(
    _comb_mm_body,
    grid=(GRID,),
    in_specs=[_p_spec, _row_spec, _dis_spec, _b_spec, _w_spec],
    out_specs=_row_spec,
    out_shape=jax.ShapeDtypeStruct((NN, F), jnp.float32),
)

_comb = pl.pallas_call(
    _comb_body,
    grid=(GRID,),
    in_specs=[_p_spec, _row_spec, _dis_spec, _b_spec],
    out_specs=_row_spec,
    out_shape=jax.ShapeDtypeStruct((NN, F), jnp.float32),
)

_final = pl.pallas_call(
    _final_body,
    grid=(GRID,),
    in_specs=[_p_spec, _row_spec, _dis_spec, _w_spec, _b_spec],
    out_specs=_row_spec,
    out_shape=jax.ShapeDtypeStruct((NN, F), jnp.float32),
)


def kernel(x, edge_index, W1, b1, W2, b2, Wmu, bmu, Wls, bls):
    src = edge_index[0]
    dst = edge_index[1]
    srcr = src.reshape(NW, NCHP, CHUNK)
    dstr = dst.reshape(NW, NCHP, CHUNK)
    e4 = jnp.stack([srcr, dstr], axis=2)

    degp = _deg_call(dst)
    d0 = degp[0, 0, :NN, None]
    d1 = degp[1, 0, :NN, None]

    g1, dis = _mm1(x, W1, d0, d1)
    P1 = _prop_call(g1, e4)
    g2 = _comb_mm(P1, g1, dis, b1[None], W2)
    P2 = _prop_call(g2, e4)
    g3 = _comb(P2, g2, dis, b2[None])
    P3 = _prop_call(g3, e4)

    Wc = jnp.concatenate([Wmu, Wls], axis=1)
    bc = jnp.concatenate([bmu, bls])[None]
    out = _final(P3, g3, dis, Wc, bc)
    return (out[:, :64], out[:, 64:])


# speedup vs baseline: 25.3575x; 1.0958x over previous
import functools

import jax
import jax.numpy as jnp
from jax import lax
from jax.experimental import pallas as pl
from jax.experimental.pallas import tpu as pltpu
from jax.experimental.pallas import tpu_sc as plsc

NN = 10000
EE = 320000
F = 128

NC = 2
NS = 16
NW = NC * NS
EPW = EE // NW
CHUNK = 80
NCHP = EPW // CHUNK
NSLAB = 31
OUT_PAD = 10240
ROWS_PT = OUT_PAD // NS
RB = 128
NCHD = EPW // CHUNK
DEG_PAD = 10240
DPT = DEG_PAD // NS


def _sc_deg_body(dst_hbm, out_hbm, ones_v, da_v, db_v, zbuf_v, deg_sp, sem):
    c = lax.axis_index("c")
    s = lax.axis_index("s")
    w = c * NS + s
    for j in range(CHUNK // 16):
        ones_v[pl.ds(j * 16, 16)] = jnp.ones((16,), jnp.float32)
    for j in range(DPT // 16):
        zbuf_v[pl.ds(j * 16, 16)] = jnp.zeros((16,), jnp.float32)
    pltpu.sync_copy(zbuf_v, deg_sp.at[pl.ds(s * DPT, DPT)])
    plsc.subcore_barrier()

    def scat(di):
        pltpu.async_copy(ones_v, deg_sp.at[di], sem, add=True)

    def scat_wait(di):
        pltpu.make_async_copy(ones_v, deg_sp.at[di], sem).wait()

    base0 = w * EPW
    pltpu.sync_copy(dst_hbm.at[pl.ds(base0, CHUNK)], da_v)
    scat(da_v)

    def body(k, carry):
        base = w * EPW + (2 * k + 1) * CHUNK
        pltpu.sync_copy(dst_hbm.at[pl.ds(base, CHUNK)], db_v)
        scat_wait(da_v)
        scat(db_v)
        pltpu.sync_copy(dst_hbm.at[pl.ds(base + CHUNK, CHUNK)], da_v)
        scat_wait(db_v)
        scat(da_v)
        return carry

    lax.fori_loop(0, (NCHD - 1) // 2, body, 0)
    scat_wait(da_v)
    plsc.subcore_barrier()
    pltpu.sync_copy(deg_sp.at[pl.ds(s * DPT, DPT)], zbuf_v)
    pltpu.sync_copy(zbuf_v, out_hbm.at[c, 0, pl.ds(s * DPT, DPT)])


def _sc_prop_body(g_hbm, e16_hbm, eL_hbm, out_hbm,
                  ea_v, pa_v, pb_v, rowsa_v, rowsb_v, zrow_v, agg_sp, gsem):
    c = lax.axis_index("c")
    s = lax.axis_index("s")
    w = c * NS + s

    def zb(i, carry):
        for j in range(F // 16):
            zrow_v[i, pl.ds(j * 16, 16)] = jnp.zeros((16,), jnp.float32)
        return carry

    lax.fori_loop(0, RB, zb, 0)
    for r in range(ROWS_PT // RB):
        pltpu.sync_copy(zrow_v, agg_sp.at[pl.ds(s * ROWS_PT + r * RB, RB)])
    plsc.subcore_barrier()

    def gather(si, rows):
        pltpu.async_copy(g_hbm.at[si], rows, gsem)

    def gather_wait(si, rows):
        pltpu.make_async_copy(g_hbm.at[si], rows, gsem).wait()

    def scat(di, rows):
        pltpu.sync_copy(rows, agg_sp.at[di], add=True)

    def do_slab(cur, nxt, k_next):
        gather(cur.at[1, 0], rowsb_v)
        gather_wait(cur.at[0, 0], rowsa_v)
        scat(cur.at[0, 1], rowsa_v)
        gather(cur.at[2, 0], rowsa_v)
        gather_wait(cur.at[1, 0], rowsb_v)
        scat(cur.at[1, 1], rowsb_v)
        gather(cur.at[3, 0], rowsb_v)
        gather_wait(cur.at[2, 0], rowsa_v)
        scat(cur.at[2, 1], rowsa_v)
        pltpu.sync_copy(e16_hbm.at[w, k_next], nxt)
        gather(nxt.at[0, 0], rowsa_v)
        gather_wait(cur.at[3, 0], rowsb_v)
        scat(cur.at[3, 1], rowsb_v)

    pltpu.sync_copy(e16_hbm.at[w, 0], pa_v)
    gather(pa_v.at[0, 0], rowsa_v)

    def body(m, carry):
        do_slab(pa_v, pb_v, 2 * m + 1)
        do_slab(pb_v, pa_v, 2 * m + 2)
        return carry

    lax.fori_loop(0, (NSLAB - 1) // 2, body, 0)
    gather(pa_v.at[1, 0], rowsb_v)
    gather_wait(pa_v.at[0, 0], rowsa_v)
    scat(pa_v.at[0, 1], rowsa_v)
    gather(pa_v.at[2, 0], rowsa_v)
    gather_wait(pa_v.at[1, 0], rowsb_v)
    scat(pa_v.at[1, 1], rowsb_v)
    gather(pa_v.at[3, 0], rowsb_v)
    gather_wait(pa_v.at[2, 0], rowsa_v)
    scat(pa_v.at[2, 1], rowsa_v)
    pltpu.sync_copy(eL_hbm.at[w], ea_v)
    gather(ea_v.at[0], rowsa_v)
    gather_wait(pa_v.at[3, 0], rowsb_v)
    scat(pa_v.at[3, 1], rowsb_v)
    gather_wait(ea_v.at[0], rowsa_v)
    scat(ea_v.at[1], rowsa_v)
    plsc.subcore_barrier()
    for r in range(ROWS_PT // RB):
        pltpu.sync_copy(agg_sp.at[pl.ds(s * ROWS_PT + r * RB, RB)], zrow_v)
        pltpu.sync_copy(zrow_v, out_hbm.at[c, pl.ds(s * ROWS_PT + r * RB, RB)])


def _sc_mesh():
    return plsc.VectorSubcoreMesh(core_axis_name="c", subcore_axis_name="s")


_deg_call = pl.kernel(
    _sc_deg_body,
    out_type=jax.ShapeDtypeStruct((NC, 1, DEG_PAD), jnp.float32),
    mesh=_sc_mesh(),
    scratch_types=[
        pltpu.VMEM((CHUNK,), jnp.float32),
        pltpu.VMEM((CHUNK,), jnp.int32),
        pltpu.VMEM((CHUNK,), jnp.int32),
        pltpu.VMEM((DPT,), jnp.float32),
        pltpu.VMEM_SHARED((DEG_PAD,), jnp.float32),
        pltpu.SemaphoreType.DMA,
    ],
)

_prop_call = pl.kernel(
    _sc_prop_body,
    out_type=jax.ShapeDtypeStruct((NC, OUT_PAD, F), jnp.float32),
    mesh=_sc_mesh(),
    scratch_types=[
        pltpu.VMEM((2, CHUNK), jnp.int32),
        pltpu.VMEM((4, 2, CHUNK), jnp.int32),
        pltpu.VMEM((4, 2, CHUNK), jnp.int32),
        pltpu.VMEM((CHUNK, F), jnp.float32),
        pltpu.VMEM((CHUNK, F), jnp.float32),
        pltpu.VMEM((RB, F), jnp.float32),
        pltpu.VMEM_SHARED((OUT_PAD, F), jnp.float32),
        pltpu.SemaphoreType.DMA,
    ],
)


BR = 2000
GRID = NN // BR


def _mm1_body(x_ref, w_ref, d0_ref, d1_ref, g_ref, dis_ref):
    dis = lax.rsqrt(d0_ref[...] + d1_ref[...] + 1.0)
    u = jnp.dot(x_ref[...], w_ref[...], preferred_element_type=jnp.float32)
    dis_ref[...] = dis
    g_ref[...] = u * dis


def _comb_mm_body(p_ref, g_ref, dis_ref, b_ref, w_ref, out_ref):
    dis = dis_ref[...]
    a = (p_ref[0] + p_ref[1] + g_ref[...]) * dis + b_ref[...]
    h = jnp.maximum(a, 0.0)
    out_ref[...] = jnp.dot(h, w_ref[...], preferred_element_type=jnp.float32) * dis


def _comb_body(p_ref, g_ref, dis_ref, b_ref, out_ref):
    dis = dis_ref[...]
    a = (p_ref[0] + p_ref[1] + g_ref[...]) * dis + b_ref[...]
    out_ref[...] = jnp.maximum(a, 0.0) * dis


def _final_body(p_ref, g_ref, dis_ref, w_ref, b_ref, out_ref):
    p = (p_ref[0] + p_ref[1] + g_ref[...]) * dis_ref[...]
    out_ref[...] = jnp.dot(p, w_ref[...], preferred_element_type=jnp.float32) + b_ref[...]


_row_spec = pl.BlockSpec((BR, F), lambda i: (i, 0))
_p_spec = pl.BlockSpec((NC, BR, F), lambda i: (0, i, 0))
_dis_spec = pl.BlockSpec((BR, 1), lambda i: (i, 0))
_w_spec = pl.BlockSpec((F, F), lambda i: (0, 0))
_b_spec = pl.BlockSpec((1, F), lambda i: (0, 0))

_mm1 = pl.pallas_call(
    _mm1_body,
    grid=(GRID,),
    in_specs=[_row_spec, _w_spec, _dis_spec, _dis_spec],
    out_specs=[_row_spec, _dis_spec],
    out_shape=[jax.ShapeDtypeStruct((NN, F), jnp.float32),
               jax.ShapeDtypeStruct((NN, 1), jnp.float32)],
)

_comb_mm = pl.pallas_call(
    _comb_mm_body,
    grid=(GRID,),
    in_specs=[_p_spec, _row_spec, _dis_spec, _b_spec, _w_spec],
    out_specs=_row_spec,
    out_shape=jax.ShapeDtypeStruct((NN, F), jnp.float32),
)

_comb = pl.pallas_call(
    _comb_body,
    grid=(GRID,),
    in_specs=[_p_spec, _row_spec, _dis_spec, _b_spec],
    out_specs=_row_spec,
    out_shape=jax.ShapeDtypeStruct((NN, F), jnp.float32),
)

_final = pl.pallas_call(
    _final_body,
    grid=(GRID,),
    in_specs=[_p_spec, _row_spec, _dis_spec, _w_spec, _b_spec],
    out_specs=_row_spec,
    out_shape=jax.ShapeDtypeStruct((NN, F), jnp.float32),
)


def kernel(x, edge_index, W1, b1, W2, b2, Wmu, bmu, Wls, bls):
    src = edge_index[0]
    dst = edge_index[1]
    srcr = src.reshape(NW, NCHP, CHUNK)
    dstr = dst.reshape(NW, NCHP, CHUNK)
    e4 = jnp.stack([srcr, dstr], axis=2)
    e16 = e4[:, :4 * NSLAB].reshape(NW, NSLAB, 4, 2, CHUNK)
    eL = e4[:, 4 * NSLAB]

    degp = _deg_call(dst)
    d0 = degp[0, 0, :NN, None]
    d1 = degp[1, 0, :NN, None]

    g1, dis = _mm1(x, W1, d0, d1)
    P1 = _prop_call(g1, e16, eL)
    g2 = _comb_mm(P1, g1, dis, b1[None], W2)
    P2 = _prop_call(g2, e16, eL)
    g3 = _comb(P2, g2, dis, b2[None])
    P3 = _prop_call(g3, e16, eL)

    Wc = jnp.concatenate([Wmu, Wls], axis=1)
    bc = jnp.concatenate([bmu, bls])[None]
    out = _final(P3, g3, dis, Wc, bc)
    return (out[:, :64], out[:, 64:])
